# baseline (speedup 1.0000x reference)
import functools

import jax
import jax.numpy as jnp
from jax import lax
from jax.experimental import pallas as pl
from jax.experimental.pallas import tpu as pltpu
from jax.experimental.pallas import tpu_sc as plsc

BATCH = 4096
SEQ = 200
HIDDEN = 128
NCLASS = 100

NC = 2
NS = 16
NW = NC * NS
B_PER_W = BATCH // NW

NBUF = 4

CH0 = 128
CH1 = SEQ - CH0

_mesh = plsc.VectorSubcoreMesh(core_axis_name="c", subcore_axis_name="s")


@functools.partial(
    pl.kernel,
    mesh=_mesh,
    out_type=jax.ShapeDtypeStruct((BATCH, HIDDEN), jnp.float32),
    scratch_types=[
        pltpu.VMEM((B_PER_W // 2, SEQ), jnp.int32),
        pltpu.VMEM((NBUF, SEQ, HIDDEN), jnp.float32),
        pltpu.VMEM((B_PER_W // 2, HIDDEN), jnp.float32),
    ]
    + [pltpu.SemaphoreType.DMA] * NBUF,
)
def _pool_sc(x_hbm, table_hbm, out_hbm, idx_v, rows_v, sums_v, *sems):
    wid = lax.axis_index("s") * NC + lax.axis_index("c")
    base = wid * B_PER_W


    def start_row(r, buf):
        pltpu.async_copy(
            table_hbm.at[idx_v.at[r, pl.ds(0, CH0)]],
            rows_v.at[buf, pl.ds(0, CH0)],
            sems[buf],
        )
        pltpu.async_copy(
            table_hbm.at[idx_v.at[r, pl.ds(CH0, CH1)]],
            rows_v.at[buf, pl.ds(CH0, CH1)],
            sems[buf],
        )

    def wait_row(buf):
        pltpu.make_async_copy(
            table_hbm.at[pl.ds(0, SEQ)], rows_v.at[buf], sems[buf]
        ).wait()

    inv = jnp.float32(1.0 / SEQ)

    def accum_row(r, buf):
        def body(s, accs):
            return tuple(
                accs[h] + rows_v[buf, s, pl.ds(h * 16, 16)] for h in range(8)
            )
        accs = lax.fori_loop(
            0, SEQ, body, tuple(jnp.zeros((16,), jnp.float32) for _ in range(8))
        )
        for h in range(8):
            sums_v[r, pl.ds(h * 16, 16)] = accs[h] * inv

    HB = B_PER_W // 2

    for half in range(2):
        pltpu.sync_copy(x_hbm.at[pl.ds(base + half * HB, HB)], idx_v)

        for j in range(NBUF):
            start_row(j, j)

        def outer(ii, carry):
            r0 = NBUF * ii
            for j in range(NBUF):
                wait_row(j)
                accum_row(r0 + j, j)

                @pl.when(r0 + j + NBUF < HB)
                def _():
                    start_row(r0 + j + NBUF, j)

            return carry

        lax.fori_loop(0, HB // NBUF, outer, 0)

        pltpu.sync_copy(sums_v, out_hbm.at[pl.ds(base + half * HB, HB)])


def _mm_body(p_ref, w_ref, b_ref, o_ref):
    o_ref[...] = (
        jnp.dot(p_ref[...], w_ref[...], preferred_element_type=jnp.float32)
        + b_ref[...]
    )


_mm = pl.pallas_call(
    _mm_body,
    grid=(8,),
    in_specs=[
        pl.BlockSpec((BATCH // 8, HIDDEN), lambda i: (i, 0)),
        pl.BlockSpec((HIDDEN, HIDDEN), lambda i: (0, 0)),
        pl.BlockSpec((1, HIDDEN), lambda i: (0, 0)),
    ],
    out_specs=pl.BlockSpec((BATCH // 8, HIDDEN), lambda i: (i, 0)),
    out_shape=jax.ShapeDtypeStruct((BATCH, HIDDEN), jnp.float32),
)


def kernel(x, table, W, b):
    pooled = _pool_sc(x, table)
    wt = jnp.pad(W, ((0, HIDDEN - NCLASS), (0, 0))).T
    bp = jnp.pad(b, (0, HIDDEN - NCLASS)).reshape(1, HIDDEN)
    return _mm(pooled, wt, bp)[:, :NCLASS]


# speedup vs baseline: 1.0346x; 1.0346x over previous
import functools

import jax
import jax.numpy as jnp
from jax import lax
from jax.experimental import pallas as pl
from jax.experimental.pallas import tpu as pltpu
from jax.experimental.pallas import tpu_sc as plsc

BATCH = 4096
SEQ = 200
HIDDEN = 128
NCLASS = 100

NC = 2
NS = 16
NW = NC * NS
B_PER_W = BATCH // NW

NBUF = 3

CH0 = 128
CH1 = SEQ - CH0

_mesh = plsc.VectorSubcoreMesh(core_axis_name="c", subcore_axis_name="s")


@functools.partial(
    pl.kernel,
    mesh=_mesh,
    out_type=jax.ShapeDtypeStruct((BATCH, HIDDEN), jnp.float32),
    scratch_types=[
        pltpu.VMEM((B_PER_W, SEQ), jnp.int32),
        pltpu.VMEM((NBUF, SEQ, HIDDEN), jnp.float32),
        pltpu.VMEM((B_PER_W, HIDDEN), jnp.float32),
    ]
    + [pltpu.SemaphoreType.DMA] * NBUF,
)
def _pool_sc(x_hbm, table_hbm, out_hbm, idx_v, rows_v, sums_v, *sems):
    wid = lax.axis_index("s") * NC + lax.axis_index("c")
    base = wid * B_PER_W

    pltpu.sync_copy(x_hbm.at[pl.ds(base, B_PER_W)], idx_v)

    def start_row(r, buf):
        pltpu.async_copy(
            table_hbm.at[idx_v.at[r, pl.ds(0, CH0)]],
            rows_v.at[buf, pl.ds(0, CH0)],
            sems[buf],
        )
        pltpu.async_copy(
            table_hbm.at[idx_v.at[r, pl.ds(CH0, CH1)]],
            rows_v.at[buf, pl.ds(CH0, CH1)],
            sems[buf],
        )

    def wait_row(buf):
        pltpu.make_async_copy(
            table_hbm.at[pl.ds(0, SEQ)], rows_v.at[buf], sems[buf]
        ).wait()

    inv = jnp.float32(1.0 / SEQ)

    def accum_row(r, buf):
        def body(s, accs):
            return tuple(
                accs[h] + rows_v[buf, s, pl.ds(h * 16, 16)] for h in range(8)
            )
        accs = lax.fori_loop(
            0, SEQ, body, tuple(jnp.zeros((16,), jnp.float32) for _ in range(8))
        )
        for h in range(8):
            sums_v[r, pl.ds(h * 16, 16)] = accs[h] * inv

    for j in range(NBUF):
        start_row(j, j)

    NFULL = B_PER_W // NBUF

    def outer(ii, carry):
        r0 = NBUF * ii
        for j in range(NBUF):
            wait_row(j)
            accum_row(r0 + j, j)

            @pl.when(r0 + j + NBUF < B_PER_W)
            def _():
                start_row(r0 + j + NBUF, j)

        return carry

    lax.fori_loop(0, NFULL, outer, 0)
    for j in range(B_PER_W - NFULL * NBUF):
        wait_row(j)
        accum_row(NFULL * NBUF + j, j)

    pltpu.sync_copy(sums_v, out_hbm.at[pl.ds(base, B_PER_W)])


def _mm_body(p_ref, w_ref, b_ref, o_ref):
    o_ref[...] = (
        jnp.dot(p_ref[...], w_ref[...], preferred_element_type=jnp.float32)
        + b_ref[...]
    )


_mm = pl.pallas_call(
    _mm_body,
    grid=(8,),
    in_specs=[
        pl.BlockSpec((BATCH // 8, HIDDEN), lambda i: (i, 0)),
        pl.BlockSpec((HIDDEN, HIDDEN), lambda i: (0, 0)),
        pl.BlockSpec((1, HIDDEN), lambda i: (0, 0)),
    ],
    out_specs=pl.BlockSpec((BATCH // 8, HIDDEN), lambda i: (i, 0)),
    out_shape=jax.ShapeDtypeStruct((BATCH, HIDDEN), jnp.float32),
)


def kernel(x, table, W, b):
    pooled = _pool_sc(x, table)
    wt = jnp.pad(W, ((0, HIDDEN - NCLASS), (0, 0))).T
    bp = jnp.pad(b, (0, HIDDEN - NCLASS)).reshape(1, HIDDEN)
    return _mm(pooled, wt, bp)[:, :NCLASS]


# speedup vs baseline: 1.0366x; 1.0019x over previous
import functools

import jax
import jax.numpy as jnp
from jax import lax
from jax.experimental import pallas as pl
from jax.experimental.pallas import tpu as pltpu
from jax.experimental.pallas import tpu_sc as plsc

BATCH = 4096
SEQ = 200
HIDDEN = 128
NCLASS = 100

NC = 2
NS = 16
NW = NC * NS
B_PER_W = BATCH // NW

NBUF = 3

CH0 = 128
CH1 = SEQ - CH0

_mesh = plsc.VectorSubcoreMesh(core_axis_name="c", subcore_axis_name="s")


@functools.partial(
    pl.kernel,
    mesh=_mesh,
    out_type=jax.ShapeDtypeStruct((BATCH, HIDDEN), jnp.float32),
    scratch_types=[
        pltpu.VMEM((B_PER_W, SEQ), jnp.int32),
        pltpu.VMEM((NBUF, SEQ, HIDDEN), jnp.float32),
        pltpu.VMEM((B_PER_W, HIDDEN), jnp.float32),
    ]
    + [pltpu.SemaphoreType.DMA] * NBUF,
)
def _pool_sc(x_hbm, table_hbm, out_hbm, idx_v, rows_v, sums_v, *sems):
    wid = lax.axis_index("s") * NC + lax.axis_index("c")
    base = wid * B_PER_W

    pltpu.sync_copy(x_hbm.at[pl.ds(base, B_PER_W)], idx_v)

    def start_row(r, buf):
        pltpu.async_copy(
            table_hbm.at[idx_v.at[r, pl.ds(0, CH0)]],
            rows_v.at[buf, pl.ds(0, CH0)],
            sems[buf],
        )
        pltpu.async_copy(
            table_hbm.at[idx_v.at[r, pl.ds(CH0, CH1)]],
            rows_v.at[buf, pl.ds(CH0, CH1)],
            sems[buf],
        )

    def wait_row(buf):
        pltpu.make_async_copy(
            table_hbm.at[pl.ds(0, SEQ)], rows_v.at[buf], sems[buf]
        ).wait()

    inv = jnp.float32(1.0 / SEQ)

    def accum_row(r, buf):
        def body(s, accs):
            return tuple(
                accs[h] + rows_v[buf, s, pl.ds(h * 16, 16)] for h in range(8)
            )
        accs = lax.fori_loop(
            0, SEQ, body, tuple(jnp.zeros((16,), jnp.float32) for _ in range(8))
        )
        for h in range(8):
            sums_v[r, pl.ds(h * 16, 16)] = accs[h] * inv

    for j in range(NBUF):
        start_row(j, j)

    NFULL = B_PER_W // NBUF

    def outer(ii, carry):
        r0 = NBUF * ii
        for j in range(NBUF):
            wait_row(j)
            accum_row(r0 + j, j)

            @pl.when(r0 + j + NBUF < B_PER_W)
            def _():
                start_row(r0 + j + NBUF, j)

        return carry

    lax.fori_loop(0, NFULL, outer, 0)
    for j in range(B_PER_W - NFULL * NBUF):
        wait_row(j)
        accum_row(NFULL * NBUF + j, j)

    pltpu.sync_copy(sums_v, out_hbm.at[pl.ds(base, B_PER_W)])


def _mm_body(p_ref, w_ref, b_ref, o_ref):
    o_ref[...] = (
        lax.dot_general(
            p_ref[...], w_ref[...],
            (((1,), (1,)), ((), ())),
            preferred_element_type=jnp.float32,
        )
        + b_ref[...]
    )


_mm = pl.pallas_call(
    _mm_body,
    grid=(8,),
    in_specs=[
        pl.BlockSpec((BATCH // 8, HIDDEN), lambda i: (i, 0)),
        pl.BlockSpec((NCLASS, HIDDEN), lambda i: (0, 0)),
        pl.BlockSpec((1, NCLASS), lambda i: (0, 0)),
    ],
    out_specs=pl.BlockSpec((BATCH // 8, NCLASS), lambda i: (i, 0)),
    out_shape=jax.ShapeDtypeStruct((BATCH, NCLASS), jnp.float32),
)


def kernel(x, table, W, b):
    pooled = _pool_sc(x, table)
    return _mm(pooled, W, b.reshape(1, NCLASS))
